# baseline (speedup 1.0000x reference)
import functools

import jax
import jax.numpy as jnp
from jax import lax
from jax.experimental import pallas as pl
from jax.experimental.pallas import tpu as pltpu
from jax.experimental.pallas import tpu_sc as plsc

N = 10000
E = 320000
F = 128
HID = 128
NCLS = 40
NCLS_P = 48
NCLS_P2 = 48

NSUB = 16
BLK = 32
ROWS_MAIN = 624
ZR = 208
TBLK = 1000


def _hp_dot(a, b):
    return jnp.dot(a, b, preferred_element_type=jnp.float32,
                   precision=lax.Precision.HIGHEST)


def _tc1_body(x_ref, w_ref, b_ref, gd_ref, sa_ref):
    y = _hp_dot(x_ref[...], w_ref[0]) + b_ref[0]
    gd_ref[0] = y
    sa_ref[0] = y[:, :F]


def _tc1(x, w_gd, b_gd):
    grid = (2, N // TBLK)
    return pl.pallas_call(
        _tc1_body,
        grid=grid,
        in_specs=[
            pl.BlockSpec((TBLK, F), lambda c, i: (i, 0)),
            pl.BlockSpec((1, F, 2 * F), lambda c, i: (c, 0, 0)),
            pl.BlockSpec((1, 1, 2 * F), lambda c, i: (c, 0, 0)),
        ],
        out_specs=[
            pl.BlockSpec((1, TBLK, 2 * F), lambda c, i: (c, i, 0)),
            pl.BlockSpec((1, TBLK, F), lambda c, i: (1 - c, i, 0)),
        ],
        out_shape=[
            jax.ShapeDtypeStruct((2, N, 2 * F), jnp.float32),
            jax.ShapeDtypeStruct((2, N, F), jnp.float32),
        ],
    )(x, w_gd, b_gd)


def _tc2_body(hio_ref, deg_ref, x_ref, gwa_ref, gwb_ref, gb1_ref, gw2_ref,
              gb2_ref, w2_ref, b2_ref, gd2_ref, sa2_ref, x2_ref):
    ind = jnp.maximum(deg_ref[0][:, 0:1], 1.0)
    outd = jnp.maximum(deg_ref[1][:, 0:1], 1.0)
    hi = hio_ref[0] / ind
    ho = hio_ref[1] / outd
    pre = jnp.maximum(_hp_dot(hi, gwa_ref[...]) + _hp_dot(ho, gwb_ref[...])
                      + gb1_ref[...], 0.0)
    z = _hp_dot(pre, gw2_ref[...]) + gb2_ref[0, 0]
    g = 1.0 / (1.0 + jnp.exp(-z))
    fused = g * hi + (1.0 - g) * ho
    x2 = jnp.maximum(fused + x_ref[...], 0.0)
    y2 = _hp_dot(x2, w2_ref[0]) + b2_ref[0]
    gd2_ref[0] = y2
    sa2_ref[0] = y2[:, :HID]
    x2_ref[...] = x2


def _tc2(h1, deg, x, gwa, gwb, gb1, gw2, gb2, w2, b2):
    W2 = HID + NCLS_P2
    grid = (2, N // TBLK)
    return pl.pallas_call(
        _tc2_body,
        grid=grid,
        in_specs=[
            pl.BlockSpec((2, TBLK, HID), lambda c, i: (0, i, 0)),
            pl.BlockSpec((2, TBLK, DW), lambda c, i: (0, i, 0)),
            pl.BlockSpec((TBLK, F), lambda c, i: (i, 0)),
            pl.BlockSpec((HID, HID), lambda c, i: (0, 0)),
            pl.BlockSpec((HID, HID), lambda c, i: (0, 0)),
            pl.BlockSpec((HID,), lambda c, i: (0,)),
            pl.BlockSpec((HID, 1), lambda c, i: (0, 0)),
            pl.BlockSpec((1, 1), lambda c, i: (0, 0)),
            pl.BlockSpec((1, HID, W2), lambda c, i: (c, 0, 0)),
            pl.BlockSpec((1, 1, W2), lambda c, i: (c, 0, 0)),
        ],
        out_specs=[
            pl.BlockSpec((1, TBLK, W2), lambda c, i: (c, i, 0)),
            pl.BlockSpec((1, TBLK, HID), lambda c, i: (1 - c, i, 0)),
            pl.BlockSpec((TBLK, HID), lambda c, i: (i, 0)),
        ],
        out_shape=[
            jax.ShapeDtypeStruct((2, N, W2), jnp.float32),
            jax.ShapeDtypeStruct((2, N, HID), jnp.float32),
            jax.ShapeDtypeStruct((N, HID), jnp.float32),
        ],
    )(h1, deg, x, gwa, gwb, gb1, gw2, gb2, w2, b2)


def _tc3_body(h2_ref, deg_ref, x2_ref, ga_ref, gb_ref, gb1_ref, gw2_ref,
              gb2_ref, rw_ref, rb_ref, out_ref):
    hin = h2_ref[0][:, :NCLS_P]
    hout = h2_ref[1][:, :NCLS_P]
    ind = jnp.maximum(deg_ref[0][:, 0:1], 1.0)
    outd = jnp.maximum(deg_ref[1][:, 0:1], 1.0)
    hi = hin / ind
    ho = hout / outd
    pre = jnp.maximum(_hp_dot(hi, ga_ref[...]) + _hp_dot(ho, gb_ref[...])
                      + gb1_ref[...], 0.0)
    z = _hp_dot(pre, gw2_ref[...]) + gb2_ref[0, 0]
    g = 1.0 / (1.0 + jnp.exp(-z))
    fused = g * hi + (1.0 - g) * ho
    res = _hp_dot(x2_ref[...], rw_ref[...]) + rb_ref[...]
    o = fused + res
    col = lax.broadcasted_iota(jnp.int32, (TBLK, NCLS_P), 1)
    neg = jnp.where(col < NCLS, o, -jnp.inf)
    m = jnp.max(neg, axis=1, keepdims=True)
    lse = jnp.log(jnp.sum(jnp.exp(neg - m), axis=1, keepdims=True))
    out_ref[...] = (o - m - lse)[:, :NCLS]


def _tc3(h2, deg, x2, ga, gb, gb1, gw2, gb2, rw, rb):
    grid = (N // TBLK,)
    return pl.pallas_call(
        _tc3_body,
        grid=grid,
        in_specs=[
            pl.BlockSpec((2, TBLK, NCLS_P2), lambda i: (0, i, 0)),
            pl.BlockSpec((2, TBLK, DW), lambda i: (0, i, 0)),
            pl.BlockSpec((TBLK, HID), lambda i: (i, 0)),
            pl.BlockSpec((NCLS_P, NCLS_P), lambda i: (0, 0)),
            pl.BlockSpec((NCLS_P, NCLS_P), lambda i: (0, 0)),
            pl.BlockSpec((NCLS_P,), lambda i: (0,)),
            pl.BlockSpec((NCLS_P, 1), lambda i: (0, 0)),
            pl.BlockSpec((1, 1), lambda i: (0, 0)),
            pl.BlockSpec((HID, NCLS_P), lambda i: (0, 0)),
            pl.BlockSpec((NCLS_P,), lambda i: (0,)),
        ],
        out_specs=pl.BlockSpec((TBLK, NCLS), lambda i: (i, 0)),
        out_shape=jax.ShapeDtypeStruct((N, NCLS), jnp.float32),
    )(h2, deg, x2, ga, gb, gb1, gw2, gb2, rw, rb)


def _make_edge_kernel(GW, SW, MW, PW, MOFF, ones_cols):
    EPT = E // NSUB
    NBLK = EPT // BLK
    SB = 25
    NSB = NBLK // SB

    def body(gd, sa, gidx, sidx, ridx, wvec, out, acc, gbuf, sbuf, msgb,
             gisb, sisb, risb, wv, sg0, sg1, sg2, ss0, ss1):
        c = lax.axis_index("c")
        s = lax.axis_index("s")
        coff = c * N
        semg = (sg0, sg1, sg2)
        sems = (ss0, ss1)

        zero16 = jnp.zeros((16,), jnp.float32)

        def zrow(r, carry):
            for kk in range(MW // 16):
                msgb[0, r, pl.ds(16 * kk, 16)] = zero16
            return carry

        lax.fori_loop(0, BLK, zrow, 0)
        abase = s * ROWS_MAIN
        for j in range(ROWS_MAIN // BLK):
            pltpu.sync_copy(msgb.at[0], acc.at[pl.ds(abase + j * BLK, BLK)])
        nfull = (ROWS_MAIN // BLK) * BLK
        if nfull < ROWS_MAIN:
            pltpu.sync_copy(msgb.at[0, pl.ds(0, ROWS_MAIN - nfull)],
                            acc.at[pl.ds(abase + nfull, ROWS_MAIN - nfull)])

        @pl.when(s == NSUB - 1)
        def _():
            pltpu.sync_copy(msgb.at[0, pl.ds(0, 16)],
                            acc.at[pl.ds(NSUB * ROWS_MAIN, 16)])

        plsc.subcore_barrier()

        pltpu.sync_copy(wvec, wv)
        ew2c = [wv[pl.ds(16 * k, 16)] for k in range(SW // 16)]
        eb2v = wv[pl.ds(SW, 16)]

        if ones_cols:
            ones16 = jnp.full((16,), 1.0, jnp.float32)
            for r in range(BLK):
                msgb[0, r, pl.ds(PW, 16)] = ones16
                msgb[1, r, pl.ds(PW, 16)] = ones16

        rowbase = c * (E // BLK) + s * NBLK

        def issue_gather(j, p):
            pltpu.async_copy(gd.at[gisb.at[j]], gbuf.at[p], semg[p])
            pltpu.async_copy(sa.at[sisb.at[j]], sbuf.at[p], semg[p])

        def wait_gather(p):
            pltpu.make_async_copy(gd.at[pl.ds(0, BLK)], gbuf.at[p],
                                  semg[p]).wait()
            pltpu.make_async_copy(sa.at[pl.ds(0, BLK)], sbuf.at[p],
                                  semg[p]).wait()

        def wait_scatter(p):
            pltpu.make_async_copy(out.at[pl.ds(0, BLK)], msgb.at[p],
                                  sems[p]).wait()

        def compute_block(pg, pm):
            @plsc.parallel_loop(0, BLK, unroll=4)
            def edge(j):
                av = eb2v
                for k in range(SW // 16):
                    h = (gbuf[pg, j, pl.ds(16 * k, 16)]
                         + sbuf[pg, j, pl.ds(16 * k, 16)])
                    av = av + jnp.maximum(h, 0.0) * ew2c[k]
                for m in (1, 2, 4, 8):
                    perm = lax.iota(jnp.int32, 16) ^ m
                    av = av + av.at[perm].get(mode="promise_in_bounds")
                sg = 1.0 / (1.0 + jnp.exp(-av))
                for k in range(PW // 16):
                    msgb[pm, j, pl.ds(16 * k, 16)] = (
                        gbuf[pg, j, pl.ds(MOFF + 16 * k, 16)] * sg)

        def sb_body(sb, carry):
            r0 = rowbase + sb * SB
            pltpu.sync_copy(gidx.at[pl.ds(r0, SB)], gisb)
            pltpu.sync_copy(sidx.at[pl.ds(r0, SB)], sisb)
            pltpu.sync_copy(ridx.at[pl.ds(r0, SB)], risb)
            issue_gather(0, 0)
            issue_gather(1, 1)

            def grp(g, gc):
                for o in range(6):
                    j = 6 * g + o

                    @pl.when(j + 2 < SB)
                    def _():
                        issue_gather(j + 2, (o + 2) % 3)

                    wait_gather(o % 3)

                    @pl.when(j >= 2)
                    def _():
                        wait_scatter(o % 2)

                    compute_block(o % 3, o % 2)
                    pltpu.async_copy(msgb.at[o % 2], acc.at[risb.at[j]],
                                     sems[o % 2], add=True)
                return gc

            lax.fori_loop(0, (SB - 1) // 6, grp, 0)

            wait_gather(0)
            wait_scatter(0)
            compute_block(0, 0)
            pltpu.async_copy(msgb.at[0], acc.at[risb.at[SB - 1]],
                             sems[0], add=True)
            wait_scatter(0)
            wait_scatter(1)
            return carry

        lax.fori_loop(0, NSB, sb_body, 0)
        plsc.subcore_barrier()

        for j in range(ROWS_MAIN // ZR):
            base = s * ROWS_MAIN + j * ZR
            pltpu.sync_copy(acc.at[pl.ds(base, ZR)],
                            out.at[pl.ds(coff + base, ZR)])

        @pl.when(s == NSUB - 1)
        def _():
            pltpu.sync_copy(acc.at[pl.ds(NSUB * ROWS_MAIN, 16)],
                            out.at[pl.ds(coff + NSUB * ROWS_MAIN, 16)])

    return functools.partial(
        pl.kernel,
        compiler_params=pltpu.CompilerParams(use_tc_tiling_on_sc=False,
                                             needs_layout_passes=False),
        out_type=jax.ShapeDtypeStruct((2 * N, MW), jnp.float32),
        mesh=plsc.VectorSubcoreMesh(core_axis_name="c", subcore_axis_name="s",
                                    num_cores=2, num_subcores=NSUB),
        scratch_types=[
            pltpu.VMEM_SHARED((N, MW), jnp.float32),
            pltpu.VMEM((3, BLK, GW), jnp.float32),
            pltpu.VMEM((3, BLK, SW), jnp.float32),
            pltpu.VMEM((2, BLK, MW), jnp.float32),
            pltpu.VMEM((SB, BLK), jnp.int32),
            pltpu.VMEM((SB, BLK), jnp.int32),
            pltpu.VMEM((SB, BLK), jnp.int32),
            pltpu.VMEM((SW + 16,), jnp.float32),
            pltpu.SemaphoreType.DMA,
            pltpu.SemaphoreType.DMA,
            pltpu.SemaphoreType.DMA,
            pltpu.SemaphoreType.DMA,
            pltpu.SemaphoreType.DMA,
        ],
    )(body)


DW = 16


@functools.lru_cache(maxsize=None)
def _deg_kernel():
    EPT = E // NSUB
    NBLK = EPT // BLK
    SB = 25
    NSB = NBLK // SB
    LAG = 4

    def body(ridx, out, dacc, onesb, risb, sem):
        c = lax.axis_index("c")
        s = lax.axis_index("s")
        coff = c * N
        zero16 = jnp.zeros((16,), jnp.float32)

        def zrow(r, carry):
            onesb[r, pl.ds(0, 16)] = zero16
            return carry

        lax.fori_loop(0, BLK, zrow, 0)
        abase = s * ROWS_MAIN
        for j in range(ROWS_MAIN // BLK):
            pltpu.sync_copy(onesb, dacc.at[pl.ds(abase + j * BLK, BLK)])
        pltpu.sync_copy(onesb.at[pl.ds(0, 16)],
                        dacc.at[pl.ds(abase + 19 * BLK, 16)])

        @pl.when(s == NSUB - 1)
        def _():
            pltpu.sync_copy(onesb.at[pl.ds(0, 16)],
                            dacc.at[pl.ds(NSUB * ROWS_MAIN, 16)])

        plsc.subcore_barrier()

        ones16 = jnp.full((16,), 1.0, jnp.float32)

        def orow(r, carry):
            onesb[r, pl.ds(0, 16)] = ones16
            return carry

        lax.fori_loop(0, BLK, orow, 0)

        def wait_deg():
            pltpu.make_async_copy(out.at[pl.ds(0, BLK)], onesb, sem).wait()

        rowbase = c * (E // BLK) + s * NBLK

        def sb_body(sb, carry):
            pltpu.sync_copy(ridx.at[pl.ds(rowbase + sb * SB, SB)], risb)
            for j in range(SB):
                if j >= LAG:
                    wait_deg()
                pltpu.async_copy(onesb, dacc.at[risb.at[j]], sem, add=True)
            for _ in range(LAG):
                wait_deg()
            return carry

        lax.fori_loop(0, NSB, sb_body, 0)
        plsc.subcore_barrier()

        for j in range(ROWS_MAIN // ZR):
            base = s * ROWS_MAIN + j * ZR
            pltpu.sync_copy(dacc.at[pl.ds(base, ZR)],
                            out.at[pl.ds(coff + base, ZR)])

        @pl.when(s == NSUB - 1)
        def _():
            pltpu.sync_copy(dacc.at[pl.ds(NSUB * ROWS_MAIN, 16)],
                            out.at[pl.ds(coff + NSUB * ROWS_MAIN, 16)])

    return functools.partial(
        pl.kernel,
        compiler_params=pltpu.CompilerParams(use_tc_tiling_on_sc=False,
                                             needs_layout_passes=False),
        out_type=jax.ShapeDtypeStruct((2 * N, DW), jnp.float32),
        mesh=plsc.VectorSubcoreMesh(core_axis_name="c", subcore_axis_name="s",
                                    num_cores=2, num_subcores=NSUB),
        scratch_types=[
            pltpu.VMEM_SHARED((N, DW), jnp.float32),
            pltpu.VMEM((BLK, DW), jnp.float32),
            pltpu.VMEM((25, BLK), jnp.int32),
            pltpu.SemaphoreType.DMA,
        ],
    )(body)


@functools.lru_cache(maxsize=None)
def _edge_l1():
    return _make_edge_kernel(GW=2 * F, SW=F, MW=HID, PW=HID,
                             MOFF=F, ones_cols=False)


@functools.lru_cache(maxsize=None)
def _edge_l2():
    return _make_edge_kernel(GW=HID + NCLS_P2, SW=HID, MW=NCLS_P2,
                             PW=NCLS_P2, MOFF=HID, ones_cols=False)


def _edge_pass_l1(gd, sa, gidx, sidx, ridx, wvec):
    return _edge_l1()(gd.reshape(2 * N, 2 * F), sa.reshape(2 * N, F),
                      gidx, sidx, ridx, wvec)


def _edge_pass_l2(gd, sa, gidx, sidx, ridx, wvec):
    return _edge_l2()(gd.reshape(2 * N, HID + NCLS_P2),
                      sa.reshape(2 * N, HID), gidx, sidx, ridx, wvec)


def _pad_cols(w, width):
    return jnp.pad(w, ((0, 0), (0, width - w.shape[-1])))


def _pad_vec(v, width):
    return jnp.pad(v, (0, width - v.shape[0]))


def _ilv(width):
    import numpy as np
    idx = np.empty((width,), dtype=np.int32)
    for g in range(width // 32):
        t = np.arange(16)
        idx[32 * g + 2 * t] = 32 * g + t
        idx[32 * g + 2 * t + 1] = 32 * g + 16 + t
    return idx


def _perm_cols(w, b, halves):
    import numpy as np
    parts, off = [], 0
    for h in halves:
        parts.append(off + _ilv(h))
        off += h
    perm = np.concatenate(parts)
    return w[:, :, perm], b[:, :, perm]


def kernel(x, edge_index, w_sd1, b_sd1, w_ds1, b_ds1, ew1_1, eb1_1, ew2_1,
           eb2_1, gw1_1, gb1_1, gw2_1, gb2_1, w_sd2, b_sd2, w_ds2, b_ds2,
           ew1_2, eb1_2, ew2_2, eb2_2, gw1_2, gb1_2, gw2_2, gb2_2,
           res_w2, res_b2):
    src = edge_index[0]
    dst = edge_index[1]
    gidx = jnp.concatenate([src, dst + N]).reshape(2 * E // BLK, BLK)
    sidx = jnp.concatenate([dst, src + N]).reshape(2 * E // BLK, BLK)
    ridx = jnp.concatenate([dst, src]).reshape(2 * E // BLK, BLK)

    w_gd1 = jnp.stack([
        jnp.concatenate([ew1_1[:F], w_sd1], axis=1),
        jnp.concatenate([ew1_1[F:], w_ds1], axis=1),
    ])
    b_gd1 = jnp.stack([
        jnp.concatenate([eb1_1, b_sd1]),
        jnp.concatenate([jnp.zeros((F,), jnp.float32), b_ds1]),
    ]).reshape(2, 1, 2 * F)
    gd1, sa1 = _tc1(x, w_gd1, b_gd1)
    deg = _deg_kernel()(ridx).reshape(2, N, DW)

    wvec1 = jnp.concatenate([ew2_1[:, 0], _pad_vec(eb2_1, 16)])
    h1 = _edge_pass_l1(gd1, sa1, gidx, sidx, ridx,
                       wvec1).reshape(2, N, HID)

    w_sd2p = _pad_cols(w_sd2, NCLS_P2)
    w_ds2p = _pad_cols(w_ds2, NCLS_P2)
    w2 = jnp.stack([
        jnp.concatenate([ew1_2[:HID], w_sd2p], axis=1),
        jnp.concatenate([ew1_2[HID:], w_ds2p], axis=1),
    ])
    b2 = jnp.stack([
        jnp.concatenate([eb1_2, _pad_vec(b_sd2, NCLS_P2)]),
        jnp.concatenate([jnp.zeros((HID,), jnp.float32),
                         _pad_vec(b_ds2, NCLS_P2)]),
    ]).reshape(2, 1, HID + NCLS_P2)
    gd2, sa2, x2 = _tc2(h1, deg, x, gw1_1[:HID], gw1_1[HID:], gb1_1, gw2_1,
                        gb2_1.reshape(1, 1), w2, b2)

    wvec2 = jnp.concatenate([ew2_2[:, 0], _pad_vec(eb2_2, 16)])
    h2 = _edge_pass_l2(gd2, sa2, gidx, sidx, ridx,
                       wvec2).reshape(2, N, NCLS_P2)

    ga = jnp.zeros((NCLS_P, NCLS_P), jnp.float32).at[:NCLS, :NCLS].set(
        gw1_2[:NCLS])
    gb = jnp.zeros((NCLS_P, NCLS_P), jnp.float32).at[:NCLS, :NCLS].set(
        gw1_2[NCLS:])
    gb1p = _pad_vec(gb1_2, NCLS_P)
    gw2p = jnp.zeros((NCLS_P, 1), jnp.float32).at[:NCLS].set(gw2_2)
    rw = _pad_cols(res_w2, NCLS_P)
    rb = _pad_vec(res_b2, NCLS_P)
    return _tc3(h2, deg, x2, ga, gb, gb1p, gw2p, gb2_2.reshape(1, 1), rw, rb)


# speedup vs baseline: 1.1235x; 1.1235x over previous
import functools

import jax
import jax.numpy as jnp
from jax import lax
from jax.experimental import pallas as pl
from jax.experimental.pallas import tpu as pltpu
from jax.experimental.pallas import tpu_sc as plsc

N = 10000
E = 320000
F = 128
HID = 128
NCLS = 40
NCLS_P = 48
NCLS_P2 = 48

NSUB = 16
BLK = 32
ROWS_MAIN = 624
ZR = 208
TBLK = 1000


def _hp_dot(a, b):
    return jnp.dot(a, b, preferred_element_type=jnp.float32,
                   precision=lax.Precision.HIGHEST)


def _tc1_body(x_ref, w_ref, b_ref, gd_ref, sa_ref):
    y = _hp_dot(x_ref[...], w_ref[0]) + b_ref[0]
    gd_ref[0] = y
    sa_ref[0] = y[:, :F]


def _tc1(x, w_gd, b_gd):
    grid = (2, N // TBLK)
    return pl.pallas_call(
        _tc1_body,
        grid=grid,
        in_specs=[
            pl.BlockSpec((TBLK, F), lambda c, i: (i, 0)),
            pl.BlockSpec((1, F, 2 * F), lambda c, i: (c, 0, 0)),
            pl.BlockSpec((1, 1, 2 * F), lambda c, i: (c, 0, 0)),
        ],
        out_specs=[
            pl.BlockSpec((1, TBLK, 2 * F), lambda c, i: (c, i, 0)),
            pl.BlockSpec((1, TBLK, F), lambda c, i: (1 - c, i, 0)),
        ],
        out_shape=[
            jax.ShapeDtypeStruct((2, N, 2 * F), jnp.float32),
            jax.ShapeDtypeStruct((2, N, F), jnp.float32),
        ],
    )(x, w_gd, b_gd)


def _tc2_body(hio_ref, x_ref, gwa_ref, gwb_ref, gb1_ref, gw2_ref, gb2_ref,
              w2_ref, b2_ref, gd2_ref, sa2_ref, x2_ref):
    hin = hio_ref[0]
    hout = hio_ref[1]
    ind = jnp.maximum(hin[:, HID:HID + 1], 1.0)
    outd = jnp.maximum(hout[:, HID:HID + 1], 1.0)
    hi = hin[:, :HID] / ind
    ho = hout[:, :HID] / outd
    pre = jnp.maximum(_hp_dot(hi, gwa_ref[...]) + _hp_dot(ho, gwb_ref[...])
                      + gb1_ref[...], 0.0)
    z = _hp_dot(pre, gw2_ref[...]) + gb2_ref[0, 0]
    g = 1.0 / (1.0 + jnp.exp(-z))
    fused = g * hi + (1.0 - g) * ho
    x2 = jnp.maximum(fused + x_ref[...], 0.0)
    y2 = _hp_dot(x2, w2_ref[0]) + b2_ref[0]
    gd2_ref[0] = y2
    sa2_ref[0] = y2[:, :HID]
    x2_ref[...] = x2


def _tc2(h1, x, gwa, gwb, gb1, gw2, gb2, w2, b2):
    W2 = HID + NCLS_P2
    grid = (2, N // TBLK)
    return pl.pallas_call(
        _tc2_body,
        grid=grid,
        in_specs=[
            pl.BlockSpec((2, TBLK, HID + NSUB), lambda c, i: (0, i, 0)),
            pl.BlockSpec((TBLK, F), lambda c, i: (i, 0)),
            pl.BlockSpec((HID, HID), lambda c, i: (0, 0)),
            pl.BlockSpec((HID, HID), lambda c, i: (0, 0)),
            pl.BlockSpec((HID,), lambda c, i: (0,)),
            pl.BlockSpec((HID, 1), lambda c, i: (0, 0)),
            pl.BlockSpec((1, 1), lambda c, i: (0, 0)),
            pl.BlockSpec((1, HID, W2), lambda c, i: (c, 0, 0)),
            pl.BlockSpec((1, 1, W2), lambda c, i: (c, 0, 0)),
        ],
        out_specs=[
            pl.BlockSpec((1, TBLK, W2), lambda c, i: (c, i, 0)),
            pl.BlockSpec((1, TBLK, HID), lambda c, i: (1 - c, i, 0)),
            pl.BlockSpec((TBLK, HID), lambda c, i: (i, 0)),
        ],
        out_shape=[
            jax.ShapeDtypeStruct((2, N, W2), jnp.float32),
            jax.ShapeDtypeStruct((2, N, HID), jnp.float32),
            jax.ShapeDtypeStruct((N, HID), jnp.float32),
        ],
    )(h1, x, gwa, gwb, gb1, gw2, gb2, w2, b2)


def _tc3_body(h2_ref, deg_ref, x2_ref, ga_ref, gb_ref, gb1_ref, gw2_ref,
              gb2_ref, rw_ref, rb_ref, out_ref):
    hin = h2_ref[0][:, :NCLS_P]
    hout = h2_ref[1][:, :NCLS_P]
    ind = jnp.maximum(deg_ref[0][:, HID:HID + 1], 1.0)
    outd = jnp.maximum(deg_ref[1][:, HID:HID + 1], 1.0)
    hi = hin / ind
    ho = hout / outd
    pre = jnp.maximum(_hp_dot(hi, ga_ref[...]) + _hp_dot(ho, gb_ref[...])
                      + gb1_ref[...], 0.0)
    z = _hp_dot(pre, gw2_ref[...]) + gb2_ref[0, 0]
    g = 1.0 / (1.0 + jnp.exp(-z))
    fused = g * hi + (1.0 - g) * ho
    res = _hp_dot(x2_ref[...], rw_ref[...]) + rb_ref[...]
    o = fused + res
    col = lax.broadcasted_iota(jnp.int32, (TBLK, NCLS_P), 1)
    neg = jnp.where(col < NCLS, o, -jnp.inf)
    m = jnp.max(neg, axis=1, keepdims=True)
    lse = jnp.log(jnp.sum(jnp.exp(neg - m), axis=1, keepdims=True))
    out_ref[...] = (o - m - lse)[:, :NCLS]


def _tc3(h2, deg, x2, ga, gb, gb1, gw2, gb2, rw, rb):
    grid = (N // TBLK,)
    return pl.pallas_call(
        _tc3_body,
        grid=grid,
        in_specs=[
            pl.BlockSpec((2, TBLK, NCLS_P2), lambda i: (0, i, 0)),
            pl.BlockSpec((2, TBLK, HID + NSUB), lambda i: (0, i, 0)),
            pl.BlockSpec((TBLK, HID), lambda i: (i, 0)),
            pl.BlockSpec((NCLS_P, NCLS_P), lambda i: (0, 0)),
            pl.BlockSpec((NCLS_P, NCLS_P), lambda i: (0, 0)),
            pl.BlockSpec((NCLS_P,), lambda i: (0,)),
            pl.BlockSpec((NCLS_P, 1), lambda i: (0, 0)),
            pl.BlockSpec((1, 1), lambda i: (0, 0)),
            pl.BlockSpec((HID, NCLS_P), lambda i: (0, 0)),
            pl.BlockSpec((NCLS_P,), lambda i: (0,)),
        ],
        out_specs=pl.BlockSpec((TBLK, NCLS), lambda i: (i, 0)),
        out_shape=jax.ShapeDtypeStruct((N, NCLS), jnp.float32),
    )(h2, deg, x2, ga, gb, gb1, gw2, gb2, rw, rb)


def _make_edge_kernel(GW, SW, MW, PW, MOFF, ones_cols):
    EPT = E // NSUB
    NBLK = EPT // BLK
    SB = 25
    NSB = NBLK // SB

    def body(gd, sa, gidx, sidx, ridx, wvec, out, acc, gbuf, sbuf, msgb,
             gisb, sisb, risb, wv, sg0, sg1, ss0, ss1):
        c = lax.axis_index("c")
        s = lax.axis_index("s")
        coff = c * N
        semg = (sg0, sg1)
        sems = (ss0, ss1)

        zero16 = jnp.zeros((16,), jnp.float32)

        def zrow(r, carry):
            for kk in range(MW // 16):
                msgb[0, r, pl.ds(16 * kk, 16)] = zero16
            return carry

        lax.fori_loop(0, BLK, zrow, 0)
        abase = s * ROWS_MAIN
        for j in range(ROWS_MAIN // BLK):
            pltpu.sync_copy(msgb.at[0], acc.at[pl.ds(abase + j * BLK, BLK)])
        nfull = (ROWS_MAIN // BLK) * BLK
        if nfull < ROWS_MAIN:
            pltpu.sync_copy(msgb.at[0, pl.ds(0, ROWS_MAIN - nfull)],
                            acc.at[pl.ds(abase + nfull, ROWS_MAIN - nfull)])

        @pl.when(s == NSUB - 1)
        def _():
            pltpu.sync_copy(msgb.at[0, pl.ds(0, 16)],
                            acc.at[pl.ds(NSUB * ROWS_MAIN, 16)])

        plsc.subcore_barrier()

        pltpu.sync_copy(wvec, wv)
        ew2c = [wv[pl.ds(16 * k, 16)] for k in range(SW // 16)]
        eb2v = wv[pl.ds(SW, 16)]

        if ones_cols:
            ones16 = jnp.full((16,), 1.0, jnp.float32)
            for r in range(BLK):
                msgb[0, r, pl.ds(PW, 16)] = ones16
                msgb[1, r, pl.ds(PW, 16)] = ones16

        rowbase = c * (E // BLK) + s * NBLK

        def issue_gather(j, p):
            pltpu.async_copy(gd.at[gisb.at[j]], gbuf.at[p], semg[p])
            pltpu.async_copy(sa.at[sisb.at[j]], sbuf.at[p], semg[p])

        def wait_gather(p):
            pltpu.make_async_copy(gd.at[pl.ds(0, BLK)], gbuf.at[p],
                                  semg[p]).wait()
            pltpu.make_async_copy(sa.at[pl.ds(0, BLK)], sbuf.at[p],
                                  semg[p]).wait()

        def wait_scatter(p):
            pltpu.make_async_copy(out.at[pl.ds(0, BLK)], msgb.at[p],
                                  sems[p]).wait()

        def compute_block(p):
            @plsc.parallel_loop(0, BLK, unroll=4)
            def edge(j):
                av = eb2v
                for k in range(SW // 16):
                    h = (gbuf[p, j, pl.ds(16 * k, 16)]
                         + sbuf[p, j, pl.ds(16 * k, 16)])
                    av = av + jnp.maximum(h, 0.0) * ew2c[k]
                for m in (1, 2, 4, 8):
                    perm = lax.iota(jnp.int32, 16) ^ m
                    av = av + av.at[perm].get(mode="promise_in_bounds")
                sg = 1.0 / (1.0 + jnp.exp(-av))
                for k in range(PW // 16):
                    msgb[p, j, pl.ds(16 * k, 16)] = (
                        gbuf[p, j, pl.ds(MOFF + 16 * k, 16)] * sg)

        def sb_body(sb, carry):
            r0 = rowbase + sb * SB
            pltpu.sync_copy(gidx.at[pl.ds(r0, SB)], gisb)
            pltpu.sync_copy(sidx.at[pl.ds(r0, SB)], sisb)
            pltpu.sync_copy(ridx.at[pl.ds(r0, SB)], risb)
            issue_gather(0, 0)

            def pair_body(pb, pcarry):
                for p in (0, 1):
                    j = 2 * pb + p

                    @pl.when(j < SB - 1)
                    def _():
                        issue_gather(j + 1, 1 - p)

                    wait_gather(p)

                    @pl.when(j >= 2)
                    def _():
                        wait_scatter(p)

                    compute_block(p)
                    pltpu.async_copy(msgb.at[p], acc.at[risb.at[j]],
                                     sems[p], add=True)
                return pcarry

            lax.fori_loop(0, SB // 2, pair_body, 0)

            wait_gather(0)
            wait_scatter(0)
            compute_block(0)
            pltpu.async_copy(msgb.at[0], acc.at[risb.at[SB - 1]],
                             sems[0], add=True)
            wait_scatter(0)
            wait_scatter(1)
            return carry

        lax.fori_loop(0, NSB, sb_body, 0)
        plsc.subcore_barrier()

        for j in range(ROWS_MAIN // ZR):
            base = s * ROWS_MAIN + j * ZR
            pltpu.sync_copy(acc.at[pl.ds(base, ZR)],
                            out.at[pl.ds(coff + base, ZR)])

        @pl.when(s == NSUB - 1)
        def _():
            pltpu.sync_copy(acc.at[pl.ds(NSUB * ROWS_MAIN, 16)],
                            out.at[pl.ds(coff + NSUB * ROWS_MAIN, 16)])

    return functools.partial(
        pl.kernel,
        compiler_params=pltpu.CompilerParams(use_tc_tiling_on_sc=False,
                                             needs_layout_passes=False),
        out_type=jax.ShapeDtypeStruct((2 * N, MW), jnp.float32),
        mesh=plsc.VectorSubcoreMesh(core_axis_name="c", subcore_axis_name="s",
                                    num_cores=2, num_subcores=NSUB),
        scratch_types=[
            pltpu.VMEM_SHARED((N, MW), jnp.float32),
            pltpu.VMEM((2, BLK, GW), jnp.float32),
            pltpu.VMEM((2, BLK, SW), jnp.float32),
            pltpu.VMEM((2, BLK, MW), jnp.float32),
            pltpu.VMEM((SB, BLK), jnp.int32),
            pltpu.VMEM((SB, BLK), jnp.int32),
            pltpu.VMEM((SB, BLK), jnp.int32),
            pltpu.VMEM((SW + 16,), jnp.float32),
            pltpu.SemaphoreType.DMA,
            pltpu.SemaphoreType.DMA,
            pltpu.SemaphoreType.DMA,
            pltpu.SemaphoreType.DMA,
        ],
    )(body)


@functools.lru_cache(maxsize=None)
def _edge_l1():
    return _make_edge_kernel(GW=2 * F, SW=F, MW=HID + NSUB, PW=HID,
                             MOFF=F, ones_cols=True)


@functools.lru_cache(maxsize=None)
def _edge_l2():
    return _make_edge_kernel(GW=HID + NCLS_P2, SW=HID, MW=NCLS_P2,
                             PW=NCLS_P2, MOFF=HID, ones_cols=False)


def _edge_pass_l1(gd, sa, gidx, sidx, ridx, wvec):
    return _edge_l1()(gd.reshape(2 * N, 2 * F), sa.reshape(2 * N, F),
                      gidx, sidx, ridx, wvec)


def _edge_pass_l2(gd, sa, gidx, sidx, ridx, wvec):
    return _edge_l2()(gd.reshape(2 * N, HID + NCLS_P2),
                      sa.reshape(2 * N, HID), gidx, sidx, ridx, wvec)


def _pad_cols(w, width):
    return jnp.pad(w, ((0, 0), (0, width - w.shape[-1])))


def _pad_vec(v, width):
    return jnp.pad(v, (0, width - v.shape[0]))


def _ilv(width):
    import numpy as np
    idx = np.empty((width,), dtype=np.int32)
    for g in range(width // 32):
        t = np.arange(16)
        idx[32 * g + 2 * t] = 32 * g + t
        idx[32 * g + 2 * t + 1] = 32 * g + 16 + t
    return idx


def _perm_cols(w, b, halves):
    import numpy as np
    parts, off = [], 0
    for h in halves:
        parts.append(off + _ilv(h))
        off += h
    perm = np.concatenate(parts)
    return w[:, :, perm], b[:, :, perm]


def kernel(x, edge_index, w_sd1, b_sd1, w_ds1, b_ds1, ew1_1, eb1_1, ew2_1,
           eb2_1, gw1_1, gb1_1, gw2_1, gb2_1, w_sd2, b_sd2, w_ds2, b_ds2,
           ew1_2, eb1_2, ew2_2, eb2_2, gw1_2, gb1_2, gw2_2, gb2_2,
           res_w2, res_b2):
    src = edge_index[0]
    dst = edge_index[1]
    gidx = jnp.concatenate([src, dst + N]).reshape(2 * E // BLK, BLK)
    sidx = jnp.concatenate([dst, src + N]).reshape(2 * E // BLK, BLK)
    ridx = jnp.concatenate([dst, src]).reshape(2 * E // BLK, BLK)

    w_gd1 = jnp.stack([
        jnp.concatenate([ew1_1[:F], w_sd1], axis=1),
        jnp.concatenate([ew1_1[F:], w_ds1], axis=1),
    ])
    b_gd1 = jnp.stack([
        jnp.concatenate([eb1_1, b_sd1]),
        jnp.concatenate([jnp.zeros((F,), jnp.float32), b_ds1]),
    ]).reshape(2, 1, 2 * F)
    gd1, sa1 = _tc1(x, w_gd1, b_gd1)

    wvec1 = jnp.concatenate([ew2_1[:, 0], _pad_vec(eb2_1, 16)])
    h1 = _edge_pass_l1(gd1, sa1, gidx, sidx, ridx,
                       wvec1).reshape(2, N, HID + NSUB)

    w_sd2p = _pad_cols(w_sd2, NCLS_P2)
    w_ds2p = _pad_cols(w_ds2, NCLS_P2)
    w2 = jnp.stack([
        jnp.concatenate([ew1_2[:HID], w_sd2p], axis=1),
        jnp.concatenate([ew1_2[HID:], w_ds2p], axis=1),
    ])
    b2 = jnp.stack([
        jnp.concatenate([eb1_2, _pad_vec(b_sd2, NCLS_P2)]),
        jnp.concatenate([jnp.zeros((HID,), jnp.float32),
                         _pad_vec(b_ds2, NCLS_P2)]),
    ]).reshape(2, 1, HID + NCLS_P2)
    gd2, sa2, x2 = _tc2(h1, x, gw1_1[:HID], gw1_1[HID:], gb1_1, gw2_1,
                        gb2_1.reshape(1, 1), w2, b2)

    wvec2 = jnp.concatenate([ew2_2[:, 0], _pad_vec(eb2_2, 16)])
    h2 = _edge_pass_l2(gd2, sa2, gidx, sidx, ridx,
                       wvec2).reshape(2, N, NCLS_P2)

    ga = jnp.zeros((NCLS_P, NCLS_P), jnp.float32).at[:NCLS, :NCLS].set(
        gw1_2[:NCLS])
    gb = jnp.zeros((NCLS_P, NCLS_P), jnp.float32).at[:NCLS, :NCLS].set(
        gw1_2[NCLS:])
    gb1p = _pad_vec(gb1_2, NCLS_P)
    gw2p = jnp.zeros((NCLS_P, 1), jnp.float32).at[:NCLS].set(gw2_2)
    rw = _pad_cols(res_w2, NCLS_P)
    rb = _pad_vec(res_b2, NCLS_P)
    return _tc3(h2, h1, x2, ga, gb, gb1p, gw2p, gb2_2.reshape(1, 1), rw, rb)
